# Initial kernel scaffold; baseline (speedup 1.0000x reference)
import jax
import jax.numpy as jnp
from jax.experimental import pallas as pl


def kernel(image, codebook):
    raise NotImplementedError("write your pallas kernel here")


# speedup vs baseline: 4.0639x; 4.0639x over previous
import functools

import jax
import jax.numpy as jnp
from jax import lax
from jax.experimental import pallas as pl
from jax.experimental.pallas import tpu as pltpu
from jax.experimental.pallas import tpu_sc as plsc

L = 1024
K = 1024
D = 256
_NC, _NS = 2, 16
_NW = _NC * _NS
_BPW = L // _NW


_ROWS = 128


def _tc_assign(x, cbt):
    def body(x_ref, cb_ref, out_ref):
        xv = x_ref[...]
        cv = cb_ref[...]
        s = lax.dot_general(
            xv, cv, (((1,), (0,)), ((), ())),
            preferred_element_type=jnp.float32,
            precision=lax.Precision.HIGHEST,
        )
        cn = jnp.sum(cv * cv, axis=0, keepdims=True)
        score = cn - 2.0 * s
        minv = jnp.min(score, axis=1, keepdims=True)
        kio = lax.broadcasted_iota(jnp.int32, (_ROWS, K), 1)
        cand = jnp.where(score == minv, kio, jnp.int32(2**30))
        out_ref[...] = jnp.min(cand, axis=1, keepdims=True)

    return pl.pallas_call(
        body,
        grid=(L // _ROWS,),
        in_specs=[
            pl.BlockSpec((_ROWS, D), lambda i: (i, 0)),
            pl.BlockSpec((D, K), lambda i: (0, 0)),
        ],
        out_specs=pl.BlockSpec((_ROWS, 1), lambda i: (i, 0)),
        out_shape=jax.ShapeDtypeStruct((L, 1), jnp.int32),
    )(x, cbt)


_sc_mesh = plsc.VectorSubcoreMesh(core_axis_name="c", subcore_axis_name="s")


@functools.partial(
    pl.kernel,
    mesh=_sc_mesh,
    out_type=jax.ShapeDtypeStruct((L, D), jnp.float32),
    scratch_types=[
        pltpu.VMEM((_BPW,), jnp.int32),
        pltpu.VMEM((_BPW, D), jnp.float32),
        pltpu.SemaphoreType.DMA,
    ],
)
def _sc_gather(table_hbm, idx_hbm, out_hbm, idx_v, rows_v, sem):
    wid = lax.axis_index("s") * _NC + lax.axis_index("c")
    base = wid * _BPW
    pltpu.sync_copy(idx_hbm.at[pl.ds(base, _BPW)], idx_v)
    pltpu.async_copy(table_hbm.at[idx_v], rows_v, sem).wait()
    pltpu.sync_copy(rows_v, out_hbm.at[pl.ds(base, _BPW)])


def kernel(image, codebook):
    x = image.reshape(32, 16, 32, 16).transpose(0, 2, 1, 3).reshape(L, D)
    x = x.reshape(64, 16, 16, 16).transpose(0, 2, 1, 3).reshape(L, D)
    cb = codebook.reshape(K, D)
    closest = _tc_assign(x, cb.T).reshape(L)
    blocks = _sc_gather(cb, closest)
    return blocks.reshape(32, 32, 16, 16, 1).transpose(0, 2, 1, 3, 4).reshape(512, 512, 1)


# speedup vs baseline: 8.3358x; 2.0512x over previous
import functools

import jax
import jax.numpy as jnp
from jax import lax
from jax.experimental import pallas as pl
from jax.experimental.pallas import tpu as pltpu
from jax.experimental.pallas import tpu_sc as plsc

L = 1024
K = 1024
D = 256
_NC, _NS = 2, 16
_NW = _NC * _NS
_ROWS = 128


def _tc_assign(image2, cbt):
    def body(img_ref, cb_ref, out_ref):
        img = img_ref[...]
        xv = jnp.concatenate(
            [img[t * 16:(t + 1) * 16, s * 256:(s + 1) * 256]
             for t in range(4) for s in range(2)],
            axis=0,
        )
        cv = cb_ref[...]
        s = lax.dot_general(
            xv, cv, (((1,), (0,)), ((), ())),
            preferred_element_type=jnp.float32,
            precision=lax.Precision.HIGHEST,
        )
        cn = jnp.sum(cv * cv, axis=0, keepdims=True)
        score = cn - 2.0 * s
        minv = jnp.min(score, axis=1, keepdims=True)
        kio = lax.broadcasted_iota(jnp.int32, (_ROWS, K), 1)
        cand = jnp.where(score == minv, kio, jnp.int32(2**30))
        out_ref[...] = jnp.min(cand, axis=1, keepdims=True)

    return pl.pallas_call(
        body,
        grid=(L // _ROWS,),
        in_specs=[
            pl.BlockSpec((64, 512), lambda q: (q, 0)),
            pl.BlockSpec((D, K), lambda q: (0, 0)),
        ],
        out_specs=pl.BlockSpec((_ROWS, 1), lambda q: (q, 0)),
        out_shape=jax.ShapeDtypeStruct((L, 1), jnp.int32),
    )(image2, cbt)


_sc_mesh = plsc.VectorSubcoreMesh(core_axis_name="c", subcore_axis_name="s")


@functools.partial(
    pl.kernel,
    mesh=_sc_mesh,
    out_type=jax.ShapeDtypeStruct((512, 512), jnp.float32),
    scratch_types=[
        pltpu.VMEM((32,), jnp.int32),
        pltpu.VMEM((32, D), jnp.float32),
        pltpu.VMEM((16, 512), jnp.float32),
        pltpu.SemaphoreType.DMA,
    ],
)
def _sc_gather(table_hbm, ids_hbm, out_hbm, ids_v, rows_v, stripe_v, sem):
    wid = lax.axis_index("s") * _NC + lax.axis_index("c")
    pltpu.sync_copy(ids_hbm.at[pl.ds(wid * 32, 32)], ids_v)
    pltpu.async_copy(table_hbm.at[ids_v], rows_v, sem).wait()
    for q in range(32):
        for i in range(16):
            stripe_v[i, pl.ds(16 * q, 16)] = rows_v[q, pl.ds(16 * i, 16)]
    pltpu.sync_copy(stripe_v, out_hbm.at[pl.ds(wid * 16, 16)])


def kernel(image, codebook):
    image2 = image.reshape(512, 512)
    cb = codebook.reshape(K, D)
    closest = _tc_assign(image2, cb.T).reshape(L)
    out = _sc_gather(cb, closest)
    return out.reshape(512, 512, 1)


# speedup vs baseline: 9.2010x; 1.1038x over previous
import functools

import jax
import jax.numpy as jnp
from jax import lax
from jax.experimental import pallas as pl
from jax.experimental.pallas import tpu as pltpu
from jax.experimental.pallas import tpu_sc as plsc

L = 1024
K = 1024
D = 256
_NC, _NS = 2, 16
_NW = _NC * _NS
_ROWS = 128


def _tc_assign(image2, cb):
    def body(img_ref, cb_ref, out_ref):
        img = img_ref[...]
        xv = jnp.concatenate(
            [img[t * 16:(t + 1) * 16, s * 256:(s + 1) * 256]
             for t in range(4) for s in range(2)],
            axis=0,
        )
        cv = cb_ref[...]
        s = lax.dot_general(
            cv, xv.T, (((1,), (0,)), ((), ())),
            preferred_element_type=jnp.float32,
            precision=lax.Precision.HIGHEST,
        )
        cn = jnp.sum(cv * cv, axis=1, keepdims=True)
        score = cn - 2.0 * s
        minv = jnp.min(score, axis=0, keepdims=True)
        kio = lax.broadcasted_iota(jnp.int32, (K, _ROWS), 0)
        cand = jnp.where(score == minv, kio, jnp.int32(2**30))
        out_ref[...] = jnp.min(cand, axis=0, keepdims=True).reshape(1, 1, _ROWS)

    return pl.pallas_call(
        body,
        grid=(L // _ROWS,),
        in_specs=[
            pl.BlockSpec((64, 512), lambda q: (q, 0)),
            pl.BlockSpec((K, D), lambda q: (0, 0)),
        ],
        out_specs=pl.BlockSpec((1, 1, _ROWS), lambda q: (q, 0, 0)),
        out_shape=jax.ShapeDtypeStruct((L // _ROWS, 1, _ROWS), jnp.int32),
    )(image2, cb)


_sc_mesh = plsc.VectorSubcoreMesh(core_axis_name="c", subcore_axis_name="s")


@functools.partial(
    pl.kernel,
    mesh=_sc_mesh,
    out_type=jax.ShapeDtypeStruct((512, 512), jnp.float32),
    scratch_types=[
        pltpu.VMEM((32,), jnp.int32),
        pltpu.VMEM((32, D), jnp.float32),
        pltpu.VMEM((16, 512), jnp.float32),
        pltpu.SemaphoreType.DMA,
    ],
)
def _sc_gather(table_hbm, ids_hbm, out_hbm, ids_v, rows_v, stripe_v, sem):
    wid = lax.axis_index("s") * _NC + lax.axis_index("c")
    pltpu.sync_copy(ids_hbm.at[pl.ds(wid * 32, 32)], ids_v)
    pltpu.async_copy(table_hbm.at[ids_v], rows_v, sem).wait()
    for q in range(32):
        for i in range(16):
            stripe_v[i, pl.ds(16 * q, 16)] = rows_v[q, pl.ds(16 * i, 16)]
    pltpu.sync_copy(stripe_v, out_hbm.at[pl.ds(wid * 16, 16)])


def kernel(image, codebook):
    image2 = image.reshape(512, 512)
    cb = codebook.reshape(K, D)
    closest = _tc_assign(image2, cb).reshape(L)
    out = _sc_gather(cb, closest)
    return out.reshape(512, 512, 1)


# speedup vs baseline: 10.3493x; 1.1248x over previous
import functools

import jax
import jax.numpy as jnp
from jax import lax
from jax.experimental import pallas as pl
from jax.experimental.pallas import tpu as pltpu
from jax.experimental.pallas import tpu_sc as plsc

L = 1024
K = 1024
D = 256
_NC, _NS = 2, 16
_NW = _NC * _NS
_ROWS = 512


def _tc_assign(image2, cb):
    def body(img_ref, cb_ref, out_ref):
        img = img_ref[...]
        xv = jnp.concatenate(
            [img[t * 16:(t + 1) * 16, s * 256:(s + 1) * 256]
             for t in range(_ROWS // 32) for s in range(2)],
            axis=0,
        )
        cv = cb_ref[...]
        s = lax.dot_general(
            cv, xv.T, (((1,), (0,)), ((), ())),
            preferred_element_type=jnp.float32,
            precision=lax.Precision.HIGHEST,
        )
        cn = jnp.sum(cv * cv, axis=1, keepdims=True)
        score = cn - 2.0 * s
        minv = jnp.min(score, axis=0, keepdims=True)
        kio = lax.broadcasted_iota(jnp.int32, (K, _ROWS), 0)
        cand = jnp.where(score == minv, kio, jnp.int32(2**30))
        out_ref[...] = jnp.min(cand, axis=0, keepdims=True).reshape(1, 1, _ROWS)

    return pl.pallas_call(
        body,
        grid=(L // _ROWS,),
        in_specs=[
            pl.BlockSpec((_ROWS // 2, 512), lambda q: (q, 0)),
            pl.BlockSpec((K, D), lambda q: (0, 0)),
        ],
        out_specs=pl.BlockSpec((1, 1, _ROWS), lambda q: (q, 0, 0)),
        out_shape=jax.ShapeDtypeStruct((L // _ROWS, 1, _ROWS), jnp.int32),
    )(image2, cb)


_sc_mesh = plsc.VectorSubcoreMesh(core_axis_name="c", subcore_axis_name="s")


@functools.partial(
    pl.kernel,
    mesh=_sc_mesh,
    out_type=jax.ShapeDtypeStruct((512, 512), jnp.float32),
    scratch_types=[
        pltpu.VMEM((32,), jnp.int32),
        pltpu.VMEM((32, D), jnp.float32),
        pltpu.VMEM((16, 512), jnp.float32),
        pltpu.SemaphoreType.DMA,
    ],
)
def _sc_gather(table_hbm, ids_hbm, out_hbm, ids_v, rows_v, stripe_v, sem):
    wid = lax.axis_index("s") * _NC + lax.axis_index("c")
    pltpu.sync_copy(ids_hbm.at[pl.ds(wid * 32, 32)], ids_v)
    pltpu.async_copy(table_hbm.at[ids_v], rows_v, sem).wait()
    for q in range(32):
        for i in range(16):
            stripe_v[i, pl.ds(16 * q, 16)] = rows_v[q, pl.ds(16 * i, 16)]
    pltpu.sync_copy(stripe_v, out_hbm.at[pl.ds(wid * 16, 16)])


def kernel(image, codebook):
    image2 = image.reshape(512, 512)
    cb = codebook.reshape(K, D)
    closest = _tc_assign(image2, cb).reshape(L)
    out = _sc_gather(cb, closest)
    return out.reshape(512, 512, 1)


# speedup vs baseline: 10.3501x; 1.0001x over previous
import functools

import jax
import jax.numpy as jnp
from jax import lax
from jax.experimental import pallas as pl
from jax.experimental.pallas import tpu as pltpu
from jax.experimental.pallas import tpu_sc as plsc

L = 1024
K = 1024
D = 256
_NC, _NS = 2, 16
_NW = _NC * _NS
_ROWS = 512


def _tc_assign(image2, cb):
    def body(img_ref, cb_ref, out_ref):
        img = img_ref[...]
        xv = jnp.concatenate(
            [img[t * 16:(t + 1) * 16, s * 256:(s + 1) * 256]
             for t in range(_ROWS // 32) for s in range(2)],
            axis=0,
        )
        cv = cb_ref[...]
        s = lax.dot_general(
            cv, xv.T, (((1,), (0,)), ((), ())),
            preferred_element_type=jnp.float32,
            precision=lax.Precision.HIGHEST,
        )
        cn = jnp.sum(cv * cv, axis=1, keepdims=True)
        score = cn - 2.0 * s
        minv = jnp.min(score, axis=0, keepdims=True)
        kio = lax.broadcasted_iota(jnp.int32, (K, _ROWS), 0)
        cand = jnp.where(score == minv, kio, jnp.int32(2**30))
        out_ref[...] = jnp.min(cand, axis=0, keepdims=True).reshape(1, 1, _ROWS)

    return pl.pallas_call(
        body,
        grid=(L // _ROWS,),
        in_specs=[
            pl.BlockSpec((_ROWS // 2, 512), lambda q: (q, 0)),
            pl.BlockSpec((K, D), lambda q: (0, 0)),
        ],
        out_specs=pl.BlockSpec((1, 1, _ROWS), lambda q: (q, 0, 0)),
        out_shape=jax.ShapeDtypeStruct((L // _ROWS, 1, _ROWS), jnp.int32),
    )(image2, cb)


_sc_mesh = plsc.VectorSubcoreMesh(core_axis_name="c", subcore_axis_name="s")


@functools.partial(
    pl.kernel,
    mesh=_sc_mesh,
    out_type=jax.ShapeDtypeStruct((512, 512), jnp.float32),
    scratch_types=[
        pltpu.VMEM((32,), jnp.int32),
        pltpu.VMEM((32, D), jnp.float32),
        pltpu.VMEM((16, 512), jnp.float32),
        pltpu.SemaphoreType.DMA,
        pltpu.SemaphoreType.DMA,
    ],
)
def _sc_gather(table_hbm, ids_hbm, out_hbm, ids_v, rows_v, stripe_v, sem0, sem1):
    wid = lax.axis_index("s") * _NC + lax.axis_index("c")
    pltpu.sync_copy(ids_hbm.at[pl.ds(wid * 32, 32)], ids_v)
    cp0 = pltpu.async_copy(
        table_hbm.at[ids_v.at[pl.ds(0, 16)]], rows_v.at[pl.ds(0, 16)], sem0)
    cp1 = pltpu.async_copy(
        table_hbm.at[ids_v.at[pl.ds(16, 16)]], rows_v.at[pl.ds(16, 16)], sem1)
    cp0.wait()
    for q in range(16):
        for i in range(16):
            stripe_v[i, pl.ds(16 * q, 16)] = rows_v[q, pl.ds(16 * i, 16)]
    cp1.wait()
    for q in range(16, 32):
        for i in range(16):
            stripe_v[i, pl.ds(16 * q, 16)] = rows_v[q, pl.ds(16 * i, 16)]
    pltpu.sync_copy(stripe_v, out_hbm.at[pl.ds(wid * 16, 16)])


def kernel(image, codebook):
    image2 = image.reshape(512, 512)
    cb = codebook.reshape(K, D)
    closest = _tc_assign(image2, cb).reshape(L)
    out = _sc_gather(cb, closest)
    return out.reshape(512, 512, 1)
